# Initial kernel scaffold; baseline (speedup 1.0000x reference)
import jax
import jax.numpy as jnp
from jax.experimental import pallas as pl


def kernel(feature, adj, W1, b1, W2, b2):
    raise NotImplementedError("write your pallas kernel here")


# speedup vs baseline: 27.6677x; 27.6677x over previous
import functools

import jax
import jax.numpy as jnp
from jax import lax
from jax.experimental import pallas as pl
from jax.experimental.pallas import tpu as pltpu
from jax.experimental.pallas import tpu_sc as plsc

N = 10000
NPAD = 10240
NC, NS = 2, 16
EPW = 10240
EPAD = NC * NS * EPW
ZW = NPAD // NS
ZROWS = 32

_MESH = plsc.VectorSubcoreMesh(
    core_axis_name="c", subcore_axis_name="s", num_cores=NC, num_subcores=NS)


def _fill_1d(ref, n, value):
  def zf(k, _):
    ref[pl.ds(k * 16, 16)] = jnp.full((16,), value, jnp.float32)
    return 0
  lax.fori_loop(0, n // 16, zf, 0)


def _fill_rows(ref, rows, width, value):
  def zf(k, _):
    r = k // (width // 16)
    col = (k % (width // 16)) * 16
    ref[r, pl.ds(col, 16)] = jnp.full((16,), value, jnp.float32)
    return 0
  lax.fori_loop(0, rows * width // 16, zf, 0)


DB = 128
DCH = EPW // DB


def _deg_body(dst_hbm, out_hbm, slab_v, ones_v, zbuf_v, deg_sh):
  c = lax.axis_index("c")
  s = lax.axis_index("s")
  _fill_1d(ones_v, DB, 1.0)
  _fill_1d(zbuf_v, ZW, 0.0)
  pltpu.sync_copy(dst_hbm.at[c, s], slab_v)
  pltpu.sync_copy(zbuf_v, deg_sh.at[pl.ds(s * ZW, ZW)])
  plsc.subcore_barrier()
  def step(j, _):
    pltpu.sync_copy(ones_v, deg_sh.at[slab_v.at[j]], add=True)
    return 0
  lax.fori_loop(0, DCH, step, 0)
  plsc.subcore_barrier()
  pltpu.sync_copy(deg_sh.at[pl.ds(s * ZW, ZW)],
                  out_hbm.at[c, pl.ds(s * ZW, ZW)])


_deg_kernel = functools.partial(
    pl.kernel,
    out_type=jax.ShapeDtypeStruct((NC, NPAD), jnp.float32),
    mesh=_MESH,
    scratch_types=[
        pltpu.VMEM((DCH, DB), jnp.int32),
        pltpu.VMEM((DB,), jnp.float32),
        pltpu.VMEM((ZW,), jnp.float32),
        pltpu.VMEM_SHARED((NPAD,), jnp.float32),
    ],
)(_deg_body)


def _make_scatter_kernel(D, B):
  CH = EPW // B
  GC = 1024 // B
  NG = CH // GC
  assert NG % 2 == 0

  def body(hs_hbm, src_hbm, dst_hbm, out_hbm,
           si0, di0, si1, di1, g0, g1, zbuf,
           isem0, isem1, gsem0, gsem1, agg_sh):
    c = lax.axis_index("c")
    s = lax.axis_index("s")
    _fill_rows(zbuf, ZROWS, D, 0.0)
    cbase = (c * NS + s) * CH
    zb = s * ZW
    def zc(t, _):
      pltpu.sync_copy(zbuf, agg_sh.at[pl.ds(zb + t * ZROWS, ZROWS)])
      return 0
    lax.fori_loop(0, ZW // ZROWS, zc, 0)
    plsc.subcore_barrier()

    def idx_start(gidx, sbuf, dbuf, sem):
      rows = pl.ds(cbase + gidx * GC, GC)
      pltpu.async_copy(src_hbm.at[rows], sbuf, sem)
      pltpu.async_copy(dst_hbm.at[rows], dbuf, sem)

    def idx_wait(sbuf, dbuf, sem):
      pltpu.make_async_copy(src_hbm.at[pl.ds(0, GC)], sbuf, sem).wait()
      pltpu.make_async_copy(dst_hbm.at[pl.ds(0, GC)], dbuf, sem).wait()

    def group(sbuf, dbuf, sem):
      idx_wait(sbuf, dbuf, sem)
      def step(i, _):
        j0 = 2 * i
        j1 = j0 + 1
        cg0 = pltpu.async_copy(hs_hbm.at[sbuf.at[j0]], g0, gsem0)
        cg1 = pltpu.async_copy(hs_hbm.at[sbuf.at[j1]], g1, gsem1)
        cg0.wait()
        pltpu.sync_copy(g0, agg_sh.at[dbuf.at[j0]], add=True)
        cg1.wait()
        pltpu.sync_copy(g1, agg_sh.at[dbuf.at[j1]], add=True)
        return 0
      lax.fori_loop(0, GC // 2, step, 0)

    idx_start(0, si0, di0, isem0)
    def pair(p, _):
      ga = 2 * p
      idx_start(ga + 1, si1, di1, isem1)
      group(si0, di0, isem0)
      idx_start(jnp.minimum(ga + 2, NG - 1), si0, di0, isem0)
      group(si1, di1, isem1)
      return 0
    lax.fori_loop(0, NG // 2, pair, 0)
    idx_wait(si0, di0, isem0)
    plsc.subcore_barrier()
    pltpu.sync_copy(agg_sh.at[pl.ds(zb, ZW)],
                    out_hbm.at[c, pl.ds(zb, ZW)])

  return functools.partial(
      pl.kernel,
      out_type=jax.ShapeDtypeStruct((NC, NPAD, D), jnp.float32),
      mesh=_MESH,
      compiler_params=pltpu.CompilerParams(use_tc_tiling_on_sc=(D == 128)),
      scratch_types=[
          pltpu.VMEM((GC, B), jnp.int32),
          pltpu.VMEM((GC, B), jnp.int32),
          pltpu.VMEM((GC, B), jnp.int32),
          pltpu.VMEM((GC, B), jnp.int32),
          pltpu.VMEM((B, D), jnp.float32),
          pltpu.VMEM((B, D), jnp.float32),
          pltpu.VMEM((ZROWS, D), jnp.float32),
          pltpu.SemaphoreType.DMA,
          pltpu.SemaphoreType.DMA,
          pltpu.SemaphoreType.DMA,
          pltpu.SemaphoreType.DMA,
          pltpu.VMEM_SHARED((NPAD, D), jnp.float32),
      ],
  )(body)


_scatter128 = _make_scatter_kernel(128, 64)
_scatter64 = _make_scatter_kernel(64, 128)


def _tc1_body(feat_ref, w1_ref, degt_ref, hs_ref, dinv_ref):
  deg = degt_ref[:, 0:1] + degt_ref[:, 1:2] + 1.0
  dinv = lax.rsqrt(deg)
  h = jnp.dot(feat_ref[...], w1_ref[...], preferred_element_type=jnp.float32)
  hs_ref[...] = dinv * h
  dinv_ref[...] = dinv


def _tc2_body(aggp_ref, hs1_ref, dinv_ref, b1_ref, w2_ref, hs2_ref):
  agg = aggp_ref[0, :, :] + aggp_ref[1, :, :] + hs1_ref[...]
  x1 = jnp.maximum(dinv_ref[...] * agg + b1_ref[...], 0.0)
  h2 = jnp.dot(x1, w2_ref[...], preferred_element_type=jnp.float32)
  hs2_ref[...] = dinv_ref[...] * h2


def _tc3_body(aggp_ref, hs2_ref, dinv_ref, b2_ref, out_ref):
  agg = aggp_ref[0, :, :] + aggp_ref[1, :, :] + hs2_ref[...]
  z = (dinv_ref[...] * agg + b2_ref[...])[:N, :]
  m = jnp.max(z, axis=1, keepdims=True)
  e = jnp.exp(z - m)
  lse = m + jnp.log(jnp.sum(e, axis=1, keepdims=True))
  out_ref[...] = z - lse


def kernel(feature, adj, W1, b1, W2, b2):
  src = adj[0]
  dst = adj[1]
  npad_edges = EPAD - src.shape[0]
  pad_src = jnp.arange(npad_edges, dtype=jnp.int32) % N
  pad_dst = jnp.arange(npad_edges, dtype=jnp.int32) % (NPAD - N) + N
  src_flat = jnp.concatenate([src, pad_src])
  dst_flat = jnp.concatenate([dst, pad_dst])

  deg_parts = _deg_kernel(dst_flat.reshape(NC, NS, DCH, DB))
  degt = deg_parts.T

  featpad = jnp.pad(feature, ((0, NPAD - N), (0, 0)))
  hs1, dinv = pl.pallas_call(
      _tc1_body,
      out_shape=[
          jax.ShapeDtypeStruct((NPAD, 128), jnp.float32),
          jax.ShapeDtypeStruct((NPAD, 1), jnp.float32),
      ],
  )(featpad, W1, degt)

  agg1 = _scatter128(hs1,
                     src_flat.reshape(-1, 64),
                     dst_flat.reshape(-1, 64))

  hs2 = pl.pallas_call(
      _tc2_body,
      out_shape=jax.ShapeDtypeStruct((NPAD, 64), jnp.float32),
  )(agg1, hs1, dinv, b1.reshape(1, -1), W2)

  agg2 = _scatter64(hs2,
                    src_flat.reshape(-1, 128),
                    dst_flat.reshape(-1, 128))

  out = pl.pallas_call(
      _tc3_body,
      out_shape=jax.ShapeDtypeStruct((N, 64), jnp.float32),
  )(agg2, hs2, dinv, b2.reshape(1, -1))
  return out


# speedup vs baseline: 32.4953x; 1.1745x over previous
import functools

import jax
import jax.numpy as jnp
from jax import lax
from jax.experimental import pallas as pl
from jax.experimental.pallas import tpu as pltpu
from jax.experimental.pallas import tpu_sc as plsc

N = 10000
NPAD = 10240
NC, NS = 2, 16
EPW = 10240
EPAD = NC * NS * EPW
ZW = NPAD // NS
ZROWS = 32

_MESH = plsc.VectorSubcoreMesh(
    core_axis_name="c", subcore_axis_name="s", num_cores=NC, num_subcores=NS)


def _fill_1d(ref, n, value):
  def zf(k, _):
    ref[pl.ds(k * 16, 16)] = jnp.full((16,), value, jnp.float32)
    return 0
  lax.fori_loop(0, n // 16, zf, 0)


def _fill_rows(ref, rows, width, value):
  def zf(k, _):
    r = k // (width // 16)
    col = (k % (width // 16)) * 16
    ref[r, pl.ds(col, 16)] = jnp.full((16,), value, jnp.float32)
    return 0
  lax.fori_loop(0, rows * width // 16, zf, 0)


DB = 128
DCH = EPW // DB


def _deg_body(dst_hbm, out_hbm, slab_v, ones_v, zbuf_v, deg_sh):
  c = lax.axis_index("c")
  s = lax.axis_index("s")
  _fill_1d(ones_v, DB, 1.0)
  _fill_1d(zbuf_v, ZW, 0.0)
  pltpu.sync_copy(dst_hbm.at[c, s], slab_v)
  pltpu.sync_copy(zbuf_v, deg_sh.at[pl.ds(s * ZW, ZW)])
  plsc.subcore_barrier()
  def step(j, _):
    pltpu.sync_copy(ones_v, deg_sh.at[slab_v.at[j]], add=True)
    return 0
  lax.fori_loop(0, DCH, step, 0)
  plsc.subcore_barrier()
  pltpu.sync_copy(deg_sh.at[pl.ds(s * ZW, ZW)],
                  out_hbm.at[c, pl.ds(s * ZW, ZW)])


_deg_kernel = functools.partial(
    pl.kernel,
    out_type=jax.ShapeDtypeStruct((NC, NPAD), jnp.float32),
    mesh=_MESH,
    scratch_types=[
        pltpu.VMEM((DCH, DB), jnp.int32),
        pltpu.VMEM((DB,), jnp.float32),
        pltpu.VMEM((ZW,), jnp.float32),
        pltpu.VMEM_SHARED((NPAD,), jnp.float32),
    ],
)(_deg_body)


def _make_scatter_kernel(D, B):
  CH = EPW // B
  GC = 1024 // B
  NG = CH // GC
  assert NG % 2 == 0

  def body(hs_hbm, src_hbm, dst_hbm, out_hbm,
           si0, di0, si1, di1, a0, a1, b0, b1, zbuf,
           isem0, isem1, as0, as1, bs0, bs1, sa0, sa1, sb0, sb1, agg_sh):
    c = lax.axis_index("c")
    s = lax.axis_index("s")
    _fill_rows(zbuf, ZROWS, D, 0.0)
    cbase = (c * NS + s) * CH
    zb = s * ZW
    def zc(t, _):
      pltpu.sync_copy(zbuf, agg_sh.at[pl.ds(zb + t * ZROWS, ZROWS)])
      return 0
    lax.fori_loop(0, ZW // ZROWS, zc, 0)
    plsc.subcore_barrier()

    def idx_start(gidx, sbuf, dbuf, sem):
      rows = pl.ds(cbase + gidx * GC, GC)
      pltpu.async_copy(src_hbm.at[rows], sbuf, sem)
      pltpu.async_copy(dst_hbm.at[rows], dbuf, sem)

    def idx_wait(sbuf, dbuf, sem):
      pltpu.make_async_copy(src_hbm.at[pl.ds(0, GC)], sbuf, sem).wait()
      pltpu.make_async_copy(dst_hbm.at[pl.ds(0, GC)], dbuf, sem).wait()

    def gather(row_ref, buf, sem):
      return pltpu.async_copy(hs_hbm.at[row_ref], buf, sem)

    def scat(buf, row_ref, sem):
      return pltpu.async_copy(buf, agg_sh.at[row_ref], sem, add=True)

    def group(sbuf, dbuf, sem):
      idx_wait(sbuf, dbuf, sem)
      gather(sbuf.at[0], a0, as0)
      gather(sbuf.at[1], a1, as1)
      def it(t, _):
        j = 4 * t
        gather(sbuf.at[j + 2], b0, bs0)
        gather(sbuf.at[j + 3], b1, bs1)
        pltpu.make_async_copy(hs_hbm.at[sbuf.at[0]], a0, as0).wait()
        ca0 = scat(a0, dbuf.at[j], sa0)
        pltpu.make_async_copy(hs_hbm.at[sbuf.at[0]], a1, as1).wait()
        ca1 = scat(a1, dbuf.at[j + 1], sa1)
        pltpu.make_async_copy(hs_hbm.at[sbuf.at[0]], b0, bs0).wait()
        cb0 = scat(b0, dbuf.at[j + 2], sb0)
        pltpu.make_async_copy(hs_hbm.at[sbuf.at[0]], b1, bs1).wait()
        cb1 = scat(b1, dbuf.at[j + 3], sb1)
        ca0.wait()
        ca1.wait()
        jn = jnp.minimum(j + 4, GC - 2)
        gather(sbuf.at[jn], a0, as0)
        gather(sbuf.at[jn + 1], a1, as1)
        cb0.wait()
        cb1.wait()
        return 0
      lax.fori_loop(0, GC // 4, it, 0)
      pltpu.make_async_copy(hs_hbm.at[sbuf.at[0]], a0, as0).wait()
      pltpu.make_async_copy(hs_hbm.at[sbuf.at[0]], a1, as1).wait()

    idx_start(0, si0, di0, isem0)
    def pair(p, _):
      ga = 2 * p
      idx_start(ga + 1, si1, di1, isem1)
      group(si0, di0, isem0)
      idx_start(jnp.minimum(ga + 2, NG - 1), si0, di0, isem0)
      group(si1, di1, isem1)
      return 0
    lax.fori_loop(0, NG // 2, pair, 0)
    idx_wait(si0, di0, isem0)
    plsc.subcore_barrier()
    pltpu.sync_copy(agg_sh.at[pl.ds(zb, ZW)],
                    out_hbm.at[c, pl.ds(zb, ZW)])

  return functools.partial(
      pl.kernel,
      out_type=jax.ShapeDtypeStruct((NC, NPAD, D), jnp.float32),
      mesh=_MESH,
      compiler_params=pltpu.CompilerParams(use_tc_tiling_on_sc=(D == 128)),
      scratch_types=[
          pltpu.VMEM((GC, B), jnp.int32),
          pltpu.VMEM((GC, B), jnp.int32),
          pltpu.VMEM((GC, B), jnp.int32),
          pltpu.VMEM((GC, B), jnp.int32),
          pltpu.VMEM((B, D), jnp.float32),
          pltpu.VMEM((B, D), jnp.float32),
          pltpu.VMEM((B, D), jnp.float32),
          pltpu.VMEM((B, D), jnp.float32),
          pltpu.VMEM((ZROWS, D), jnp.float32),
          pltpu.SemaphoreType.DMA,
          pltpu.SemaphoreType.DMA,
          pltpu.SemaphoreType.DMA,
          pltpu.SemaphoreType.DMA,
          pltpu.SemaphoreType.DMA,
          pltpu.SemaphoreType.DMA,
          pltpu.SemaphoreType.DMA,
          pltpu.SemaphoreType.DMA,
          pltpu.SemaphoreType.DMA,
          pltpu.SemaphoreType.DMA,
          pltpu.VMEM_SHARED((NPAD, D), jnp.float32),
      ],
  )(body)


_scatter128 = _make_scatter_kernel(128, 64)
_scatter64 = _make_scatter_kernel(64, 128)


def _tc1_body(feat_ref, w1_ref, degt_ref, hs_ref, dinv_ref):
  deg = degt_ref[:, 0:1] + degt_ref[:, 1:2] + 1.0
  dinv = lax.rsqrt(deg)
  h = jnp.dot(feat_ref[...], w1_ref[...], preferred_element_type=jnp.float32)
  hs_ref[...] = dinv * h
  dinv_ref[...] = dinv


def _tc2_body(aggp_ref, hs1_ref, dinv_ref, b1_ref, w2_ref, hs2_ref):
  agg = aggp_ref[0, :, :] + aggp_ref[1, :, :] + hs1_ref[...]
  x1 = jnp.maximum(dinv_ref[...] * agg + b1_ref[...], 0.0)
  h2 = jnp.dot(x1, w2_ref[...], preferred_element_type=jnp.float32)
  hs2_ref[...] = dinv_ref[...] * h2


def _tc3_body(aggp_ref, hs2_ref, dinv_ref, b2_ref, out_ref):
  agg = aggp_ref[0, :, :] + aggp_ref[1, :, :] + hs2_ref[...]
  z = (dinv_ref[...] * agg + b2_ref[...])[:N, :]
  m = jnp.max(z, axis=1, keepdims=True)
  e = jnp.exp(z - m)
  lse = m + jnp.log(jnp.sum(e, axis=1, keepdims=True))
  out_ref[...] = z - lse


def kernel(feature, adj, W1, b1, W2, b2):
  src = adj[0]
  dst = adj[1]
  npad_edges = EPAD - src.shape[0]
  pad_src = jnp.arange(npad_edges, dtype=jnp.int32) % N
  pad_dst = jnp.arange(npad_edges, dtype=jnp.int32) % (NPAD - N) + N
  src_flat = jnp.concatenate([src, pad_src])
  dst_flat = jnp.concatenate([dst, pad_dst])

  deg_parts = _deg_kernel(dst_flat.reshape(NC, NS, DCH, DB))
  degt = deg_parts.T

  featpad = jnp.pad(feature, ((0, NPAD - N), (0, 0)))
  hs1, dinv = pl.pallas_call(
      _tc1_body,
      out_shape=[
          jax.ShapeDtypeStruct((NPAD, 128), jnp.float32),
          jax.ShapeDtypeStruct((NPAD, 1), jnp.float32),
      ],
  )(featpad, W1, degt)

  agg1 = _scatter128(hs1,
                     src_flat.reshape(-1, 64),
                     dst_flat.reshape(-1, 64))

  hs2 = pl.pallas_call(
      _tc2_body,
      out_shape=jax.ShapeDtypeStruct((NPAD, 64), jnp.float32),
  )(agg1, hs1, dinv, b1.reshape(1, -1), W2)

  agg2 = _scatter64(hs2,
                    src_flat.reshape(-1, 128),
                    dst_flat.reshape(-1, 128))

  out = pl.pallas_call(
      _tc3_body,
      out_shape=jax.ShapeDtypeStruct((N, 64), jnp.float32),
  )(agg2, hs2, dinv, b2.reshape(1, -1))
  return out
